# Initial kernel scaffold; baseline (speedup 1.0000x reference)
import jax
import jax.numpy as jnp
from jax.experimental import pallas as pl


def kernel(idx0_batch, idx1_batch, w1, fc2_w, fc2_b, fc3_w, fc3_b, fc4_w, fc4_b):
    raise NotImplementedError("write your pallas kernel here")


# speedup vs baseline: 2.4447x; 2.4447x over previous
import functools

import jax
import jax.numpy as jnp
from jax import lax
from jax.experimental import pallas as pl
from jax.experimental.pallas import tpu as pltpu
from jax.experimental.pallas import tpu_sc as plsc

_TABLE = 40960
_H = 256
_B = 16384
_L = 20

_NC = 2
_NS = 16
_NW = _NC * _NS
_NSEG = 2 * _B
_SEG_PER_W = _NSEG // _NW
_CHUNK = 4
_NCHUNK = _SEG_PER_W // _CHUNK
_LANES = 16


def _sc_gather_sum(table, idx_flat):
    mesh = plsc.VectorSubcoreMesh(core_axis_name="c", subcore_axis_name="s")

    @functools.partial(
        pl.kernel,
        out_type=jax.ShapeDtypeStruct((_NSEG, _H), jnp.float32),
        mesh=mesh,
        scratch_types=[
            pltpu.VMEM((_SEG_PER_W * _L,), jnp.int32),
            pltpu.VMEM((_CHUNK * _L, _H), jnp.float32),
            pltpu.VMEM((_CHUNK, _H), jnp.float32),
            pltpu.SemaphoreType.DMA,
        ],
    )
    def k(table_hbm, idx_hbm, out_hbm, idx_v, rows_v, out_v, sem):
        wid = lax.axis_index("s") * _NC + lax.axis_index("c")
        idx_base = wid * (_SEG_PER_W * _L)
        seg_base = wid * _SEG_PER_W
        pltpu.sync_copy(idx_hbm.at[pl.ds(idx_base, _SEG_PER_W * _L)], idx_v)

        def chunk_body(g, _):
            pltpu.async_copy(
                table_hbm.at[idx_v.at[pl.ds(g * (_CHUNK * _L), _CHUNK * _L)]],
                rows_v, sem).wait()
            for c in range(_CHUNK):
                for hh in range(_H // _LANES):
                    sl = pl.ds(hh * _LANES, _LANES)
                    acc = rows_v[c * _L, sl]
                    for r in range(1, _L):
                        acc = acc + rows_v[c * _L + r, sl]
                    out_v[c, sl] = jnp.maximum(acc, 0.0)
            pltpu.sync_copy(out_v, out_hbm.at[pl.ds(seg_base + g * _CHUNK,
                                                    _CHUNK)])
            return ()

        lax.fori_loop(0, _NCHUNK, chunk_body, (), unroll=False)

    return k(table, idx_flat)


def _mlp_body(h_ref, w2_ref, b2_ref, w3_ref, b3_ref, w4_ref, b4_ref, out_ref):
    h = h_ref[...]
    z = jnp.maximum(
        jnp.dot(h, w2_ref[...], preferred_element_type=jnp.float32)
        + b2_ref[...], 0.0)
    z = jnp.maximum(
        jnp.dot(z, w3_ref[...], preferred_element_type=jnp.float32)
        + b3_ref[...], 0.0)
    out_ref[...] = jnp.sum(z * w4_ref[...], axis=1) + b4_ref[0, 0]


def _mlp(h, fc2_w, fc2_b, fc3_w, fc3_b, fc4_w, fc4_b):
    blk = 2048
    grid = (_B // blk,)
    full = lambda *s: pl.BlockSpec(s, lambda i: (0,) * len(s))
    return pl.pallas_call(
        _mlp_body,
        grid=grid,
        in_specs=[
            pl.BlockSpec((blk, 2 * _H), lambda i: (i, 0)),
            full(2 * _H, 32), full(1, 32),
            full(32, 32), full(1, 32),
            full(1, 32), full(1, 1),
        ],
        out_specs=pl.BlockSpec((blk,), lambda i: (i,)),
        out_shape=jax.ShapeDtypeStruct((_B,), jnp.float32),
    )(h, fc2_w.T, fc2_b.reshape(1, 32), fc3_w.T, fc3_b.reshape(1, 32),
      fc4_w.reshape(1, 32), fc4_b.reshape(1, 1))


def kernel(idx0_batch, idx1_batch, w1, fc2_w, fc2_b, fc3_w, fc3_b,
           fc4_w, fc4_b):
    table = w1.reshape(2 * _TABLE, _H)
    idx_all = jnp.stack([idx0_batch, idx1_batch + _TABLE], axis=1)
    idx_flat = idx_all.reshape(-1)
    h = _sc_gather_sum(table, idx_flat)
    h2 = h.reshape(_B, 2 * _H)
    return _mlp(h2, fc2_w, fc2_b, fc3_w, fc3_b, fc4_w, fc4_b)


# speedup vs baseline: 3.9022x; 1.5962x over previous
import functools

import jax
import jax.numpy as jnp
from jax import lax
from jax.experimental import pallas as pl
from jax.experimental.pallas import tpu as pltpu
from jax.experimental.pallas import tpu_sc as plsc

_TABLE = 40960
_H = 256
_B = 16384
_L = 20

_NC = 2
_NS = 16
_NW = _NC * _NS
_NSEG = 2 * _B
_SEG_PER_W = _NSEG // _NW
_CHUNK = 4
_NCHUNK = _SEG_PER_W // _CHUNK
_LANES = 16


def _sc_gather_sum(table, idx_flat):
    mesh = plsc.VectorSubcoreMesh(core_axis_name="c", subcore_axis_name="s")

    nhalf = _NCHUNK // 2

    @functools.partial(
        pl.kernel,
        out_type=jax.ShapeDtypeStruct((_NSEG, _H), jnp.float32),
        mesh=mesh,
        scratch_types=[
            pltpu.VMEM((_SEG_PER_W * _L,), jnp.int32),
            pltpu.VMEM((_CHUNK * _L, _H), jnp.float32),
            pltpu.VMEM((_CHUNK * _L, _H), jnp.float32),
            pltpu.VMEM((_CHUNK, _H), jnp.float32),
            pltpu.VMEM((_CHUNK, _H), jnp.float32),
            pltpu.SemaphoreType.DMA,
            pltpu.SemaphoreType.DMA,
            pltpu.SemaphoreType.DMA,
            pltpu.SemaphoreType.DMA,
        ],
    )
    def k(table_hbm, idx_hbm, out_hbm, idx_v, rows_a, rows_b, out_a, out_b,
          sem_a, sem_b, osem_a, osem_b):
        wid = lax.axis_index("s") * _NC + lax.axis_index("c")
        idx_base = wid * (_SEG_PER_W * _L)
        seg_base = wid * _SEG_PER_W
        pltpu.sync_copy(idx_hbm.at[pl.ds(idx_base, _SEG_PER_W * _L)], idx_v)

        def issue_gather(g, rows, sem):
            pltpu.async_copy(
                table_hbm.at[idx_v.at[pl.ds(g * (_CHUNK * _L), _CHUNK * _L)]],
                rows, sem)

        def wait_gather(rows, sem):
            pltpu.make_async_copy(table_hbm.at[pl.ds(0, _CHUNK * _L)],
                                  rows, sem).wait()

        def accumulate(rows, out):
            for c in range(_CHUNK):
                for hh in range(_H // _LANES):
                    sl = pl.ds(hh * _LANES, _LANES)
                    acc = rows[c * _L, sl]
                    for r in range(1, _L):
                        acc = acc + rows[c * _L + r, sl]
                    out[c, sl] = jnp.maximum(acc, 0.0)

        def out_slice(g):
            return out_hbm.at[pl.ds(seg_base + g * _CHUNK, _CHUNK)]

        def half(j, g, rows, out, sem, osem):
            wait_gather(rows, sem)

            @pl.when(j > 0)
            def _():
                pltpu.make_async_copy(out, out_slice(0), osem).wait()

            accumulate(rows, out)

            @pl.when(j < nhalf - 1)
            def _():
                issue_gather(g + 2, rows, sem)

            pltpu.async_copy(out, out_slice(g), osem)

        issue_gather(0, rows_a, sem_a)
        issue_gather(1, rows_b, sem_b)

        def body(j, _):
            half(j, 2 * j, rows_a, out_a, sem_a, osem_a)
            half(j, 2 * j + 1, rows_b, out_b, sem_b, osem_b)
            return ()

        lax.fori_loop(0, nhalf, body, (), unroll=False)
        pltpu.make_async_copy(out_a, out_slice(0), osem_a).wait()
        pltpu.make_async_copy(out_b, out_slice(0), osem_b).wait()

    return k(table, idx_flat)


def _mlp_body(h_ref, w2_ref, b2_ref, w3_ref, b3_ref, w4_ref, b4_ref, out_ref):
    h = h_ref[...]
    z = jnp.maximum(
        jnp.dot(h, w2_ref[...], preferred_element_type=jnp.float32)
        + b2_ref[...], 0.0)
    z = jnp.maximum(
        jnp.dot(z, w3_ref[...], preferred_element_type=jnp.float32)
        + b3_ref[...], 0.0)
    out_ref[...] = jnp.sum(z * w4_ref[...], axis=1) + b4_ref[0, 0]


def _mlp(h, fc2_w, fc2_b, fc3_w, fc3_b, fc4_w, fc4_b):
    blk = 2048
    grid = (_B // blk,)
    full = lambda *s: pl.BlockSpec(s, lambda i: (0,) * len(s))
    return pl.pallas_call(
        _mlp_body,
        grid=grid,
        in_specs=[
            pl.BlockSpec((blk, 2 * _H), lambda i: (i, 0)),
            full(2 * _H, 32), full(1, 32),
            full(32, 32), full(1, 32),
            full(1, 32), full(1, 1),
        ],
        out_specs=pl.BlockSpec((blk,), lambda i: (i,)),
        out_shape=jax.ShapeDtypeStruct((_B,), jnp.float32),
    )(h, fc2_w.T, fc2_b.reshape(1, 32), fc3_w.T, fc3_b.reshape(1, 32),
      fc4_w.reshape(1, 32), fc4_b.reshape(1, 1))


def kernel(idx0_batch, idx1_batch, w1, fc2_w, fc2_b, fc3_w, fc3_b,
           fc4_w, fc4_b):
    table = w1.reshape(2 * _TABLE, _H)
    idx_all = jnp.stack([idx0_batch, idx1_batch + _TABLE], axis=1)
    idx_flat = idx_all.reshape(-1)
    h = _sc_gather_sum(table, idx_flat)
    h2 = h.reshape(_B, 2 * _H)
    return _mlp(h2, fc2_w, fc2_b, fc3_w, fc3_b, fc4_w, fc4_b)
